# baseline (speedup 1.0000x reference)
import functools

import jax
import jax.numpy as jnp
from jax import lax
from jax.experimental import pallas as pl
from jax.experimental.pallas import tpu as pltpu
from jax.experimental.pallas import tpu_sc as plsc

N_NODES = 10000
D_FEAT = 128
N_EDGES = 320000

NC = 2
NS = 16
HALF = N_NODES // NC
EPT = N_EDGES // NS
EC = 224
NCH = 90
LAST_LO = 10
ACC_R = 5008
TRASH = HALF
K = 32
DR = 312
SROWS = 624

_mesh = plsc.VectorSubcoreMesh(core_axis_name="c", subcore_axis_name="s")


@functools.partial(
    pl.kernel,
    mesh=_mesh,
    out_type=jax.ShapeDtypeStruct((N_NODES, D_FEAT), jnp.float32),
    scratch_types=[
        pltpu.VMEM_SHARED((ACC_R, D_FEAT), jnp.float32),
        pltpu.VMEM_SHARED((N_NODES, D_FEAT), jnp.float32),
        pltpu.VMEM((256,), jnp.int32),
        pltpu.VMEM((256,), jnp.int32),
        pltpu.VMEM((8, K), jnp.int32),
        pltpu.VMEM((8, K), jnp.int32),
        pltpu.VMEM((K, D_FEAT), jnp.float32),
        pltpu.VMEM((K, D_FEAT), jnp.float32),
        pltpu.VMEM((D_FEAT,), jnp.float32),
        pltpu.SemaphoreType.DMA,
        pltpu.SemaphoreType.DMA,
        pltpu.SemaphoreType.DMA,
        pltpu.SemaphoreType.DMA,
    ],
    compiler_params=pltpu.CompilerParams(needs_layout_passes=False),
)
def _gcn_sc(feat, srcv, dstv, w, out, acc, ftab, sstage, dstage, gsrc, gdst,
            rows0, rows1, wv, sem0, sem1, esem, fsem):
    c = lax.axis_index("c")
    s = lax.axis_index("s")
    zero16 = jnp.zeros((16,), jnp.float32)

    ft = pltpu.async_copy(feat.at[pl.ds(s * SROWS, SROWS)],
                          ftab.at[pl.ds(s * SROWS, SROWS)], fsem)
    pltpu.sync_copy(w, wv)

    for r in range(K):
        for j in range(8):
            rows0[r, pl.ds(j * 16, 16)] = zero16
    r0 = s * DR
    for kk in range(9):
        pltpu.sync_copy(rows0, acc.at[pl.ds(r0 + kk * K, K)])
    pltpu.sync_copy(rows0.at[pl.ds(0, 24)], acc.at[pl.ds(r0 + 288, 24)])

    @pl.when(s == NS - 1)
    def _ztail():
        pltpu.sync_copy(rows0.at[pl.ds(0, 16)],
                        acc.at[pl.ds(NS * DR, ACC_R - NS * DR)])

    e0 = pltpu.async_copy(srcv.at[pl.ds(s * EPT, EC)],
                          sstage.at[pl.ds(0, EC)], esem)
    e1 = pltpu.async_copy(dstv.at[pl.ds(s * EPT, EC)],
                          dstage.at[pl.ds(0, EC)], esem)

    ft.wait()

    @pl.when(s == NS - 1)
    def _ftail():
        pltpu.sync_copy(feat.at[pl.ds(NS * SROWS, N_NODES - NS * SROWS)],
                        ftab.at[pl.ds(NS * SROWS, N_NODES - NS * SROWS)])

    e0.wait()
    e1.wait()

    plsc.subcore_barrier()

    base_node = c * HALF
    lanes = lax.iota(jnp.int32, 16)
    ones16 = jnp.full((16,), True)
    trash16 = jnp.full((16,), TRASH, jnp.int32)
    zeros16i = jnp.zeros((16,), jnp.int32)

    def chunk_body(ch, _):
        last = ch == NCH - 1
        lo = jnp.where(last, LAST_LO, 0)

        def fb(i, off):
            sl = sstage[pl.ds(i * 16, 16)]
            dl = dstage[pl.ds(i * 16, 16)] - base_node
            m = (dl >= 0) & (dl < HALF)
            mi = m.astype(jnp.int32)
            pos = off + jnp.cumsum(mi) - 1
            plsc.store_scatter(gsrc, [pos >> 5, pos & 31], sl, mask=m)
            plsc.store_scatter(gdst, [pos >> 5, pos & 31], dl, mask=m)
            return off + jnp.sum(mi)

        cnt = lax.fori_loop(lo, EC // 16, fb, jnp.int32(0))

        @pl.when(ch + 1 < NCH)
        def _():
            eb = s * EPT + jnp.where(ch + 1 == NCH - 1,
                                     EPT - EC, (ch + 1) * EC)
            pltpu.async_copy(srcv.at[pl.ds(eb, EC)],
                             sstage.at[pl.ds(0, EC)], esem)
            pltpu.async_copy(dstv.at[pl.ds(eb, EC)],
                             dstage.at[pl.ds(0, EC)], esem)

        for i in range(2):
            pidx = cnt + i * 16 + lanes
            plsc.store_scatter(gsrc, [pidx >> 5, pidx & 31], zeros16i,
                               mask=ones16)
            plsc.store_scatter(gdst, [pidx >> 5, pidx & 31], trash16,
                               mask=ones16)

        nw = (cnt + 31) >> 5
        npr = nw >> 1

        def pair(p, _):
            g0 = pltpu.async_copy(ftab.at[gsrc.at[2 * p]], rows0, sem0)
            g1 = pltpu.async_copy(ftab.at[gsrc.at[2 * p + 1]], rows1, sem1)
            g0.wait()
            pltpu.sync_copy(rows0, acc.at[gdst.at[2 * p]], add=True)
            g1.wait()
            pltpu.sync_copy(rows1, acc.at[gdst.at[2 * p + 1]], add=True)
            return 0

        lax.fori_loop(0, npr, pair, 0)

        @pl.when((nw & 1) == 1)
        def _odd():
            g = pltpu.async_copy(ftab.at[gsrc.at[2 * npr]], rows0, sem0)
            g.wait()
            pltpu.sync_copy(rows0, acc.at[gdst.at[2 * npr]], add=True)

        @pl.when(ch + 1 < NCH)
        def _w():
            pltpu.make_async_copy(srcv.at[pl.ds(0, EC)],
                                  sstage.at[pl.ds(0, EC)], esem).wait()
            pltpu.make_async_copy(dstv.at[pl.ds(0, EC)],
                                  dstage.at[pl.ds(0, EC)], esem).wait()

        return 0

    lax.fori_loop(0, NCH, chunk_body, 0)

    plsc.subcore_barrier()

    wregs = [wv[pl.ds(j * 16, 16)] for j in range(8)]

    def scale_rows(nr):
        def mbody(r, _):
            for j in range(8):
                rows0[r, pl.ds(j * 16, 16)] = rows0[r, pl.ds(j * 16, 16)] * wregs[j]
            return 0
        lax.fori_loop(0, nr, mbody, 0, unroll=2)

    for kk in range(9):
        rs = r0 + kk * K
        pltpu.sync_copy(acc.at[pl.ds(rs, K)], rows0)
        scale_rows(K)
        pltpu.sync_copy(rows0, out.at[pl.ds(c * HALF + rs, K)])
    pltpu.sync_copy(acc.at[pl.ds(r0 + 288, 24)], rows0.at[pl.ds(0, 24)])
    scale_rows(24)
    pltpu.sync_copy(rows0.at[pl.ds(0, 24)],
                    out.at[pl.ds(c * HALF + r0 + 288, 24)])

    @pl.when(s == NS - 1)
    def _tail():
        pltpu.sync_copy(acc.at[pl.ds(NS * DR, 8)], rows0.at[pl.ds(0, 8)])
        scale_rows(8)
        pltpu.sync_copy(rows0.at[pl.ds(0, 8)],
                        out.at[pl.ds(c * HALF + NS * DR, 8)])


def kernel(features, edge_index, W):
    src = edge_index[0]
    dst = edge_index[1]
    return _gcn_sc(features, src, dst, W)


# speedup vs baseline: 1.0325x; 1.0325x over previous
import functools

import jax
import jax.numpy as jnp
from jax import lax
from jax.experimental import pallas as pl
from jax.experimental.pallas import tpu as pltpu
from jax.experimental.pallas import tpu_sc as plsc

N_NODES = 10000
D_FEAT = 128
N_EDGES = 320000

NC = 2
NS = 16
HALF = N_NODES // NC
EPT = N_EDGES // NS
EC = 224
NCH = 90
LAST_LO = 10
ACC_R = 5008
TRASH = HALF
K = 32
DR = 312
SROWS = 624

_mesh = plsc.VectorSubcoreMesh(core_axis_name="c", subcore_axis_name="s")


@functools.partial(
    pl.kernel,
    mesh=_mesh,
    out_type=jax.ShapeDtypeStruct((N_NODES, D_FEAT), jnp.float32),
    scratch_types=[
        pltpu.VMEM_SHARED((ACC_R, D_FEAT), jnp.float32),
        pltpu.VMEM_SHARED((N_NODES, D_FEAT), jnp.float32),
        pltpu.VMEM((256,), jnp.int32),
        pltpu.VMEM((256,), jnp.int32),
        pltpu.VMEM((8, K), jnp.int32),
        pltpu.VMEM((8, K), jnp.int32),
        pltpu.VMEM((K, D_FEAT), jnp.float32),
        pltpu.VMEM((K, D_FEAT), jnp.float32),
        pltpu.VMEM((D_FEAT,), jnp.float32),
        pltpu.SemaphoreType.DMA,
        pltpu.SemaphoreType.DMA,
        pltpu.SemaphoreType.DMA,
        pltpu.SemaphoreType.DMA,
        pltpu.SemaphoreType.DMA,
        pltpu.SemaphoreType.DMA,
    ],
    compiler_params=pltpu.CompilerParams(needs_layout_passes=False),
)
def _gcn_sc(feat, srcv, dstv, w, out, acc, ftab, sstage, dstage, gsrc, gdst,
            rows0, rows1, wv, sem0, sem1, sem2, sem3, esem, fsem):
    c = lax.axis_index("c")
    s = lax.axis_index("s")
    zero16 = jnp.zeros((16,), jnp.float32)

    ft = pltpu.async_copy(feat.at[pl.ds(s * SROWS, SROWS)],
                          ftab.at[pl.ds(s * SROWS, SROWS)], fsem)
    pltpu.sync_copy(w, wv)

    for r in range(K):
        for j in range(8):
            rows0[r, pl.ds(j * 16, 16)] = zero16
    r0 = s * DR
    for kk in range(9):
        pltpu.sync_copy(rows0, acc.at[pl.ds(r0 + kk * K, K)])
    pltpu.sync_copy(rows0.at[pl.ds(0, 24)], acc.at[pl.ds(r0 + 288, 24)])

    @pl.when(s == NS - 1)
    def _ztail():
        pltpu.sync_copy(rows0.at[pl.ds(0, 16)],
                        acc.at[pl.ds(NS * DR, ACC_R - NS * DR)])

    e0 = pltpu.async_copy(srcv.at[pl.ds(s * EPT, EC)],
                          sstage.at[pl.ds(0, EC)], esem)
    e1 = pltpu.async_copy(dstv.at[pl.ds(s * EPT, EC)],
                          dstage.at[pl.ds(0, EC)], esem)

    ft.wait()

    @pl.when(s == NS - 1)
    def _ftail():
        pltpu.sync_copy(feat.at[pl.ds(NS * SROWS, N_NODES - NS * SROWS)],
                        ftab.at[pl.ds(NS * SROWS, N_NODES - NS * SROWS)])

    e0.wait()
    e1.wait()

    plsc.subcore_barrier()

    base_node = c * HALF
    lanes = lax.iota(jnp.int32, 16)
    ones16 = jnp.full((16,), True)
    trash16 = jnp.full((16,), TRASH, jnp.int32)
    zeros16i = jnp.zeros((16,), jnp.int32)

    def chunk_body(ch, _):
        last = ch == NCH - 1
        lo = jnp.where(last, LAST_LO, 0)

        def fb(i, off):
            sl = sstage[pl.ds(i * 16, 16)]
            dl = dstage[pl.ds(i * 16, 16)] - base_node
            m = (dl >= 0) & (dl < HALF)
            mi = m.astype(jnp.int32)
            pos = off + jnp.cumsum(mi) - 1
            plsc.store_scatter(gsrc, [pos >> 5, pos & 31], sl, mask=m)
            plsc.store_scatter(gdst, [pos >> 5, pos & 31], dl, mask=m)
            return off + jnp.sum(mi)

        cnt = lax.fori_loop(lo, EC // 16, fb, jnp.int32(0))

        @pl.when(ch + 1 < NCH)
        def _():
            eb = s * EPT + jnp.where(ch + 1 == NCH - 1,
                                     EPT - EC, (ch + 1) * EC)
            pltpu.async_copy(srcv.at[pl.ds(eb, EC)],
                             sstage.at[pl.ds(0, EC)], esem)
            pltpu.async_copy(dstv.at[pl.ds(eb, EC)],
                             dstage.at[pl.ds(0, EC)], esem)

        for i in range(2):
            pidx = cnt + i * 16 + lanes
            plsc.store_scatter(gsrc, [pidx >> 5, pidx & 31], zeros16i,
                               mask=ones16)
            plsc.store_scatter(gdst, [pidx >> 5, pidx & 31], trash16,
                               mask=ones16)

        nw = (cnt + 31) >> 5
        npr = nw >> 1

        @pl.when(npr > 0)
        def _pair0():
            g0 = pltpu.async_copy(ftab.at[gsrc.at[0]], rows0, sem0)
            g1 = pltpu.async_copy(ftab.at[gsrc.at[1]], rows1, sem1)
            g0.wait()
            pltpu.async_copy(rows0, acc.at[gdst.at[0]], sem2, add=True)
            g1.wait()
            pltpu.async_copy(rows1, acc.at[gdst.at[1]], sem3, add=True)

        def pair(p, _):
            pltpu.make_async_copy(rows0, acc.at[gdst.at[0]], sem2).wait()
            g0 = pltpu.async_copy(ftab.at[gsrc.at[2 * p]], rows0, sem0)
            pltpu.make_async_copy(rows1, acc.at[gdst.at[0]], sem3).wait()
            g1 = pltpu.async_copy(ftab.at[gsrc.at[2 * p + 1]], rows1, sem1)
            g0.wait()
            pltpu.async_copy(rows0, acc.at[gdst.at[2 * p]], sem2, add=True)
            g1.wait()
            pltpu.async_copy(rows1, acc.at[gdst.at[2 * p + 1]], sem3, add=True)
            return 0

        lax.fori_loop(1, npr, pair, 0)

        @pl.when((nw & 1) == 1)
        def _odd():
            @pl.when(npr > 0)
            def _w0():
                pltpu.make_async_copy(rows0, acc.at[gdst.at[0]], sem2).wait()
            g = pltpu.async_copy(ftab.at[gsrc.at[2 * npr]], rows0, sem0)
            g.wait()
            pltpu.async_copy(rows0, acc.at[gdst.at[2 * npr]], sem2, add=True)

        @pl.when(nw > 0)
        def _dr0():
            pltpu.make_async_copy(rows0, acc.at[gdst.at[0]], sem2).wait()

        @pl.when(npr > 0)
        def _dr1():
            pltpu.make_async_copy(rows1, acc.at[gdst.at[0]], sem3).wait()

        @pl.when(ch + 1 < NCH)
        def _w():
            pltpu.make_async_copy(srcv.at[pl.ds(0, EC)],
                                  sstage.at[pl.ds(0, EC)], esem).wait()
            pltpu.make_async_copy(dstv.at[pl.ds(0, EC)],
                                  dstage.at[pl.ds(0, EC)], esem).wait()

        return 0

    lax.fori_loop(0, NCH, chunk_body, 0)

    plsc.subcore_barrier()

    wregs = [wv[pl.ds(j * 16, 16)] for j in range(8)]

    def scale_rows(nr):
        def mbody(r, _):
            for j in range(8):
                rows0[r, pl.ds(j * 16, 16)] = rows0[r, pl.ds(j * 16, 16)] * wregs[j]
            return 0
        lax.fori_loop(0, nr, mbody, 0, unroll=2)

    for kk in range(9):
        rs = r0 + kk * K
        pltpu.sync_copy(acc.at[pl.ds(rs, K)], rows0)
        scale_rows(K)
        pltpu.sync_copy(rows0, out.at[pl.ds(c * HALF + rs, K)])
    pltpu.sync_copy(acc.at[pl.ds(r0 + 288, 24)], rows0.at[pl.ds(0, 24)])
    scale_rows(24)
    pltpu.sync_copy(rows0.at[pl.ds(0, 24)],
                    out.at[pl.ds(c * HALF + r0 + 288, 24)])

    @pl.when(s == NS - 1)
    def _tail():
        pltpu.sync_copy(acc.at[pl.ds(NS * DR, 8)], rows0.at[pl.ds(0, 8)])
        scale_rows(8)
        pltpu.sync_copy(rows0.at[pl.ds(0, 8)],
                        out.at[pl.ds(c * HALF + NS * DR, 8)])


def kernel(features, edge_index, W):
    src = edge_index[0]
    dst = edge_index[1]
    return _gcn_sc(features, src, dst, W)
